# baseline (speedup 1.0000x reference)
import jax
import jax.numpy as jnp
from jax import lax
from jax.experimental import pallas as pl
from jax.experimental.pallas import tpu as pltpu
from jax.experimental.pallas import tpu_sc as plsc

HS = 64
L = 200
NROWS = L + 2


def _fm_body(items_hbm, feat_hbm, user_hbm, bias_hbm, uidx_hbm, pref_hbm,
             res_hbm, mat_hbm,
             idx_v, uidx_v, rows_v, bias_v, res_v, sem):
    wid = lax.axis_index("s") * 2 + lax.axis_index("c")

    @pl.when(wid == 0)
    def _():
        pltpu.sync_copy(pref_hbm.at[0], idx_v.at[pl.ds(0, L)])
        pltpu.sync_copy(uidx_hbm.at[0], uidx_v.at[pl.ds(0, 2)])
        pltpu.sync_copy(bias_hbm, bias_v.at[pl.ds(0, 1)])

        copies = [pltpu.async_copy(user_hbm.at[0], rows_v.at[0], sem)]
        item_idx = uidx_v[pl.ds(0, 16)][1]
        copies.append(pltpu.async_copy(items_hbm.at[item_idx],
                                       rows_v.at[1], sem))
        for c in range(13):
            vec = idx_v[pl.ds(16 * c, 16)]
            for lane in range(16):
                r = 16 * c + lane
                if r >= L:
                    break
                copies.append(pltpu.async_copy(feat_hbm.at[vec[lane]],
                                               rows_v.at[2 + r], sem))
        for cp in copies:
            cp.wait()

        def body(i, acc):
            return tuple(acc[c] + rows_v[i, pl.ds(c * 16, 16)] for c in range(4))

        zero = jnp.zeros((16,), jnp.float32)
        p = lax.fori_loop(2, NROWS, body, (zero, zero, zero, zero))

        total = bias_v[pl.ds(0, 16)][0]
        for c in range(4):
            ue = rows_v[0, pl.ds(c * 16, 16)]
            ie = rows_v[1, pl.ds(c * 16, 16)]
            total = total + jnp.sum(ue * ie + (ue + ie) * p[c])
        res_v[pl.ds(0, 16)] = jnp.full((16,), total, jnp.float32)

        pltpu.sync_copy(res_v.at[pl.ds(0, 1)], res_hbm)
        pltpu.sync_copy(rows_v, mat_hbm)


def kernel(items_emb, feature_emb, user_emb, Bias, ui_pair, feature_index,
           preference_index):
    del feature_index
    mesh = plsc.VectorSubcoreMesh(core_axis_name="c", subcore_axis_name="s")
    fn = pl.kernel(
        _fm_body,
        mesh=mesh,
        out_type=(
            jax.ShapeDtypeStruct((1,), jnp.float32),
            jax.ShapeDtypeStruct((NROWS, HS + 1), jnp.float32),
        ),
        scratch_types=[
            pltpu.VMEM((208,), jnp.int32),
            pltpu.VMEM((16,), jnp.int32),
            pltpu.VMEM((NROWS, HS + 1), jnp.float32),
            pltpu.VMEM((16,), jnp.float32),
            pltpu.VMEM((16,), jnp.float32),
            pltpu.SemaphoreType.DMA,
        ],
        compiler_params=pltpu.CompilerParams(needs_layout_passes=False),
    )
    res, mat = fn(items_emb, feature_emb, user_emb, Bias, ui_pair,
                  preference_index)
    result = res.reshape(1, 1)
    fb = mat[None, :, HS:]
    nz = mat[None, :, :HS]
    return (result, fb, nz)


# speedup vs baseline: 1.0091x; 1.0091x over previous
import jax
import jax.numpy as jnp
from jax import lax
from jax.experimental import pallas as pl
from jax.experimental.pallas import tpu as pltpu
from jax.experimental.pallas import tpu_sc as plsc

HS = 64
L = 200
NROWS = L + 2


def _fm_body(items_hbm, feat_hbm, user_hbm, bias_hbm, uidx_hbm, pref_hbm,
             res_hbm, mat_hbm,
             idx_v, uidx_v, rows_v, bias_v, res_v, sem):
    wid = lax.axis_index("s") * 2 + lax.axis_index("c")

    @pl.when(wid == 0)
    def _():
        pltpu.sync_copy(pref_hbm.at[0], idx_v.at[pl.ds(0, L)])
        pltpu.sync_copy(uidx_hbm.at[0], uidx_v.at[pl.ds(0, 2)])
        pltpu.sync_copy(bias_hbm, bias_v.at[pl.ds(0, 1)])

        copies = [pltpu.async_copy(user_hbm.at[0], rows_v.at[0], sem)]
        item_idx = uidx_v[pl.ds(0, 16)][1]
        copies.append(pltpu.async_copy(items_hbm.at[item_idx],
                                       rows_v.at[1], sem))
        for c in range(1):
            vec = idx_v[pl.ds(16 * c, 16)]
            for lane in range(16):
                r = 16 * c + lane
                if r >= L:
                    break
                copies.append(pltpu.async_copy(feat_hbm.at[vec[lane]],
                                               rows_v.at[2 + r], sem))
        for cp in copies:
            cp.wait()

        def body(i, acc):
            return tuple(acc[c] + rows_v[i, pl.ds(c * 16, 16)] for c in range(4))

        zero = jnp.zeros((16,), jnp.float32)
        p = lax.fori_loop(2, 18, body, (zero, zero, zero, zero))

        total = bias_v[pl.ds(0, 16)][0]
        for c in range(4):
            ue = rows_v[0, pl.ds(c * 16, 16)]
            ie = rows_v[1, pl.ds(c * 16, 16)]
            total = total + jnp.sum(ue * ie + (ue + ie) * p[c])
        res_v[pl.ds(0, 16)] = jnp.full((16,), total, jnp.float32)

        pltpu.sync_copy(res_v.at[pl.ds(0, 1)], res_hbm)
        pltpu.sync_copy(rows_v, mat_hbm)


def kernel(items_emb, feature_emb, user_emb, Bias, ui_pair, feature_index,
           preference_index):
    del feature_index
    mesh = plsc.VectorSubcoreMesh(core_axis_name="c", subcore_axis_name="s",
                                  num_cores=1)
    fn = pl.kernel(
        _fm_body,
        mesh=mesh,
        out_type=(
            jax.ShapeDtypeStruct((1,), jnp.float32),
            jax.ShapeDtypeStruct((NROWS, HS + 1), jnp.float32),
        ),
        scratch_types=[
            pltpu.VMEM((208,), jnp.int32),
            pltpu.VMEM((16,), jnp.int32),
            pltpu.VMEM((NROWS, HS + 1), jnp.float32),
            pltpu.VMEM((16,), jnp.float32),
            pltpu.VMEM((16,), jnp.float32),
            pltpu.SemaphoreType.DMA,
        ],
        compiler_params=pltpu.CompilerParams(
            needs_layout_passes=False, skip_device_barrier=True),
    )
    res, mat = fn(items_emb, feature_emb, user_emb, Bias, ui_pair,
                  preference_index)
    result = res.reshape(1, 1)
    fb = mat[None, :, HS:]
    nz = mat[None, :, :HS]
    return (result, fb, nz)


# speedup vs baseline: 8.2143x; 8.1401x over previous
import jax
import jax.numpy as jnp
from jax import lax
from jax.experimental import pallas as pl
from jax.experimental.pallas import tpu as pltpu
from jax.experimental.pallas import tpu_sc as plsc

HS = 64
HS1 = HS + 1
L = 200
V_FEAT = 100000
V_ITEM = 1000000
NT = 16
SLOTS = 5
MAT_W = 256
PREF0 = 16
SPARE0 = 72


def _fm_body(featT_hbm, itemsT_hbm, user_hbm, bias_hbm, ui_hbm, pref_hbm,
             mat_hbm, res_hbm,
             idx_v, uiv, ubuf, ibuf, rowstage, rowbuf, ctile_v, sall_v,
             bias_v, res_v, sem):
    wid = lax.axis_index("s")
    iota = lax.iota(jnp.int32, 16)

    pltpu.sync_copy(pref_hbm.at[0], idx_v.at[pl.ds(0, L)])
    c12 = idx_v[pl.ds(192, 16)]
    idx_v[pl.ds(192, 16)] = jnp.where(lax.iota(jnp.int32, 16) < 8, c12, 0)
    pltpu.sync_copy(ui_hbm.at[0], uiv.at[pl.ds(0, 2)])
    pltpu.sync_copy(user_hbm.at[0], ubuf.at[pl.ds(0, HS1)])
    i_item = uiv[pl.ds(0, 16)][1]
    woff = (i_item // 64) * 64
    lane_it = i_item - woff

    acc = jnp.zeros((16,), jnp.float32)
    res_v[pl.ds(0, 16)] = acc

    for slot in range(SLOTS):
        d = wid + NT * slot

        @pl.when(d < HS1)
        def _():
            pltpu.sync_copy(featT_hbm.at[d], rowstage)
            pltpu.sync_copy(itemsT_hbm.at[d, pl.ds(woff, 64)], ibuf)
            ue = jnp.float32(0.0)
            for k in range(5):
                uc = ubuf[pl.ds(16 * k, 16)]
                ue = ue + jnp.sum(jnp.where(iota + (16 * k) == d, uc, 0.0))
            ie = jnp.float32(0.0)
            for k in range(4):
                ic = ibuf[pl.ds(16 * k, 16)]
                ie = ie + jnp.sum(jnp.where(iota + (16 * k) == lane_it,
                                            ic, 0.0))
            hv = (jnp.where(iota == 0, ue, 0.0)
                  + jnp.where(iota == 1, ie, 0.0))
            rowbuf[pl.ds(0, 16)] = hv
            psum = jnp.float32(0.0)
            for c in range(13):
                idxc = idx_v[pl.ds(16 * c, 16)]
                g = plsc.load_gather(rowstage, [idxc])
                valid = iota + (16 * c) < L
                psum = psum + jnp.sum(jnp.where(valid, g, 0.0))
                rowbuf[pl.ds(PREF0 + 16 * c, 16)] = g
            pltpu.sync_copy(rowbuf, mat_hbm.at[d])
            contrib = jnp.where(d < HS, ue * ie + (ue + ie) * psum, 0.0)
            cur = res_v[pl.ds(0, 16)]
            res_v[pl.ds(0, 16)] = cur + jnp.where(iota == 0, contrib, 0.0)

    ctile_v[pl.ds(0, 16)] = res_v[pl.ds(0, 16)]
    zero16 = jnp.zeros((16,), jnp.float32)
    for k in range(1, MAT_W // 16):
        ctile_v[pl.ds(16 * k, 16)] = zero16
    pltpu.sync_copy(ctile_v, mat_hbm.at[wid + SPARE0])
    plsc.subcore_barrier()

    @pl.when(wid == 0)
    def _():
        pltpu.sync_copy(bias_hbm, bias_v.at[pl.ds(0, 1)])
        pltpu.sync_copy(mat_hbm.at[pl.ds(SPARE0, NT)], sall_v)
        tot = jnp.zeros((16,), jnp.float32)
        for s in range(NT):
            tot = tot + sall_v[s, pl.ds(0, 16)]
        total = tot[0] + bias_v[pl.ds(0, 16)][0]
        res_v[pl.ds(0, 16)] = jnp.full((16,), total, jnp.float32)
        pltpu.sync_copy(res_v.at[pl.ds(0, 1)], res_hbm)


def kernel(items_emb, feature_emb, user_emb, Bias, ui_pair, feature_index,
           preference_index):
    del feature_index
    mesh = plsc.VectorSubcoreMesh(core_axis_name="c", subcore_axis_name="s",
                                  num_cores=1)
    fn = pl.kernel(
        _fm_body,
        mesh=mesh,
        out_type=(
            jax.ShapeDtypeStruct((SPARE0 + NT, MAT_W), jnp.float32),
            jax.ShapeDtypeStruct((1,), jnp.float32),
        ),
        scratch_types=[
            pltpu.VMEM((208,), jnp.int32),
            pltpu.VMEM((16,), jnp.int32),
            pltpu.VMEM((80,), jnp.float32),
            pltpu.VMEM((64,), jnp.float32),
            pltpu.VMEM((V_FEAT,), jnp.float32),
            pltpu.VMEM((MAT_W,), jnp.float32),
            pltpu.VMEM((MAT_W,), jnp.float32),
            pltpu.VMEM((NT, MAT_W), jnp.float32),
            pltpu.VMEM((16,), jnp.float32),
            pltpu.VMEM((16,), jnp.float32),
            pltpu.SemaphoreType.DMA,
        ],
        compiler_params=pltpu.CompilerParams(needs_layout_passes=False),
    )
    matT, res = fn(feature_emb.T, items_emb.T, user_emb, Bias, ui_pair,
                   preference_index)
    result = res.reshape(1, 1)
    matT = matT[:HS1]
    mat = jnp.concatenate(
        [matT[:, 0:2], matT[:, PREF0:PREF0 + L]], axis=1).T
    fb = mat[None, :, HS:]
    nz = mat[None, :, :HS]
    return (result, fb, nz)


# speedup vs baseline: 9.1038x; 1.1083x over previous
import jax
import jax.numpy as jnp
from jax import lax
from jax.experimental import pallas as pl
from jax.experimental.pallas import tpu as pltpu
from jax.experimental.pallas import tpu_sc as plsc

HS = 64
HS1 = HS + 1
L = 200
V_FEAT = 100000
V_ITEM = 1000000
NT = 16
SLOTS = 5
MAT_W = 256
PREF0 = 16
SPARE0 = 72


def _fm_body(featT_hbm, itemsT_hbm, user_hbm, bias_hbm, ui_hbm, pref_hbm,
             mat_hbm, res_hbm,
             idx_v, uiv, ubuf, ibuf, rowstage, rowbuf, ctile_v, sall_v,
             bias_v, res_v, sem):
    core = lax.axis_index("c")
    wid = lax.axis_index("s")
    iota = lax.iota(jnp.int32, 16)

    pltpu.sync_copy(pref_hbm.at[0], idx_v.at[pl.ds(0, L)])
    c12 = idx_v[pl.ds(192, 16)]
    idx_v[pl.ds(192, 16)] = jnp.where(lax.iota(jnp.int32, 16) < 8, c12, 0)
    pltpu.sync_copy(ui_hbm.at[0], uiv.at[pl.ds(0, 2)])
    pltpu.sync_copy(user_hbm.at[0], ubuf.at[pl.ds(0, HS1)])
    i_item = uiv[pl.ds(0, 16)][1]
    woff = (i_item // 64) * 64
    lane_it = i_item - woff

    acc = jnp.zeros((16,), jnp.float32)
    res_v[pl.ds(0, 16)] = acc

    def process(d, with_contrib):
        h1 = pltpu.async_copy(featT_hbm.at[d], rowstage, sem)
        pltpu.sync_copy(itemsT_hbm.at[d, pl.ds(woff, 64)], ibuf)
        ue = jnp.float32(0.0)
        for k in range(5):
            uc = ubuf[pl.ds(16 * k, 16)]
            ue = ue + jnp.sum(jnp.where(iota + (16 * k) == d, uc, 0.0))
        ie = jnp.float32(0.0)
        for k in range(4):
            ic = ibuf[pl.ds(16 * k, 16)]
            ie = ie + jnp.sum(jnp.where(iota + (16 * k) == lane_it, ic, 0.0))
        hv = (jnp.where(iota == 0, ue, 0.0)
              + jnp.where(iota == 1, ie, 0.0))
        rowbuf[pl.ds(0, 16)] = hv
        h1.wait()
        psum = jnp.float32(0.0)
        for c in range(13):
            idxc = idx_v[pl.ds(16 * c, 16)]
            g = plsc.load_gather(rowstage, [idxc])
            valid = iota + (16 * c) < L
            psum = psum + jnp.sum(jnp.where(valid, g, 0.0))
            rowbuf[pl.ds(PREF0 + 16 * c, 16)] = g
        pltpu.sync_copy(rowbuf, mat_hbm.at[d])
        if with_contrib:
            contrib = ue * ie + (ue + ie) * psum
            cur = res_v[pl.ds(0, 16)]
            res_v[pl.ds(0, 16)] = cur + jnp.where(iota == 0, contrib, 0.0)

    @pl.when(core == 0)
    def _():
        for slot in range(4):
            process(wid + NT * slot, True)
        ctile_v[pl.ds(0, 16)] = res_v[pl.ds(0, 16)]
        zero16 = jnp.zeros((16,), jnp.float32)
        for k in range(1, MAT_W // 16):
            ctile_v[pl.ds(16 * k, 16)] = zero16
        pltpu.sync_copy(ctile_v, mat_hbm.at[wid + SPARE0])

    @pl.when((core == 1) & (wid == 0))
    def _():
        process(wid + HS, False)

    plsc.subcore_barrier()

    @pl.when((core == 0) & (wid == 0))
    def _():
        pltpu.sync_copy(bias_hbm, bias_v.at[pl.ds(0, 1)])
        pltpu.sync_copy(mat_hbm.at[pl.ds(SPARE0, NT)], sall_v)
        tot = jnp.zeros((16,), jnp.float32)
        for s in range(NT):
            tot = tot + sall_v[s, pl.ds(0, 16)]
        total = tot[0] + bias_v[pl.ds(0, 16)][0]
        res_v[pl.ds(0, 16)] = jnp.full((16,), total, jnp.float32)
        pltpu.sync_copy(res_v.at[pl.ds(0, 1)], res_hbm)


def kernel(items_emb, feature_emb, user_emb, Bias, ui_pair, feature_index,
           preference_index):
    del feature_index
    mesh = plsc.VectorSubcoreMesh(core_axis_name="c", subcore_axis_name="s",
                                  num_cores=2)
    fn = pl.kernel(
        _fm_body,
        mesh=mesh,
        out_type=(
            jax.ShapeDtypeStruct((SPARE0 + NT, MAT_W), jnp.float32),
            jax.ShapeDtypeStruct((1,), jnp.float32),
        ),
        scratch_types=[
            pltpu.VMEM((208,), jnp.int32),
            pltpu.VMEM((16,), jnp.int32),
            pltpu.VMEM((80,), jnp.float32),
            pltpu.VMEM((64,), jnp.float32),
            pltpu.VMEM((V_FEAT,), jnp.float32),
            pltpu.VMEM((MAT_W,), jnp.float32),
            pltpu.VMEM((MAT_W,), jnp.float32),
            pltpu.VMEM((NT, MAT_W), jnp.float32),
            pltpu.VMEM((16,), jnp.float32),
            pltpu.VMEM((16,), jnp.float32),
            pltpu.SemaphoreType.DMA,
        ],
        compiler_params=pltpu.CompilerParams(needs_layout_passes=False),
    )
    matT, res = fn(feature_emb.T, items_emb.T, user_emb, Bias, ui_pair,
                   preference_index)
    result = res.reshape(1, 1)
    matT = matT[:HS1]
    mat = jnp.concatenate(
        [matT[:, 0:2], matT[:, PREF0:PREF0 + L]], axis=1).T
    fb = mat[None, :, HS:]
    nz = mat[None, :, :HS]
    return (result, fb, nz)


# speedup vs baseline: 9.2257x; 1.0134x over previous
import jax
import jax.numpy as jnp
from jax import lax
from jax.experimental import pallas as pl
from jax.experimental.pallas import tpu as pltpu
from jax.experimental.pallas import tpu_sc as plsc

HS = 64
HS1 = HS + 1
L = 200
V_FEAT = 100000
V_ITEM = 1000000
NT = 16
SLOTS = 5
MAT_W = 256
PREF0 = 16
SPARE0 = 72


def _fm_body(featT_hbm, itemsT_hbm, user_hbm, bias_hbm, ui_hbm, pref_hbm,
             mat_hbm, res_hbm,
             idx_v, uiv, ubuf, ibuf, rowstage, rowbuf, ctile_v, sall_v,
             bias_v, res_v, sem):
    core = lax.axis_index("c")
    wid = lax.axis_index("s")
    iota = lax.iota(jnp.int32, 16)

    pltpu.sync_copy(pref_hbm.at[0], idx_v.at[pl.ds(0, L)])
    c12 = idx_v[pl.ds(192, 16)]
    idx_v[pl.ds(192, 16)] = jnp.where(lax.iota(jnp.int32, 16) < 8, c12, 0)
    pltpu.sync_copy(ui_hbm.at[0], uiv.at[pl.ds(0, 2)])
    pltpu.sync_copy(user_hbm.at[0], ubuf.at[pl.ds(0, HS1)])
    i_item = uiv[pl.ds(0, 16)][1]
    woff = (i_item // 64) * 64
    lane_it = i_item - woff

    acc = jnp.zeros((16,), jnp.float32)
    res_v[pl.ds(0, 16)] = acc

    def process(d, with_contrib):
        h1 = pltpu.async_copy(featT_hbm.at[d], rowstage, sem)
        pltpu.sync_copy(itemsT_hbm.at[d, pl.ds(woff, 64)], ibuf)
        ue = jnp.float32(0.0)
        for k in range(5):
            uc = ubuf[pl.ds(16 * k, 16)]
            ue = ue + jnp.sum(jnp.where(iota + (16 * k) == d, uc, 0.0))
        ie = jnp.float32(0.0)
        for k in range(4):
            ic = ibuf[pl.ds(16 * k, 16)]
            ie = ie + jnp.sum(jnp.where(iota + (16 * k) == lane_it, ic, 0.0))
        hv = (jnp.where(iota == 14, ue, 0.0)
              + jnp.where(iota == 15, ie, 0.0))
        rowbuf[pl.ds(0, 16)] = hv
        h1.wait()
        psum = jnp.float32(0.0)
        for c in range(13):
            idxc = idx_v[pl.ds(16 * c, 16)]
            g = plsc.load_gather(rowstage, [idxc])
            valid = iota + (16 * c) < L
            psum = psum + jnp.sum(jnp.where(valid, g, 0.0))
            rowbuf[pl.ds(PREF0 + 16 * c, 16)] = g
        pltpu.sync_copy(rowbuf, mat_hbm.at[d])
        if with_contrib:
            contrib = ue * ie + (ue + ie) * psum
            cur = res_v[pl.ds(0, 16)]
            res_v[pl.ds(0, 16)] = cur + jnp.where(iota == 0, contrib, 0.0)

    @pl.when(core == 0)
    def _():
        for slot in range(4):
            process(wid + NT * slot, True)
        ctile_v[pl.ds(0, 16)] = res_v[pl.ds(0, 16)]
        zero16 = jnp.zeros((16,), jnp.float32)
        for k in range(1, MAT_W // 16):
            ctile_v[pl.ds(16 * k, 16)] = zero16
        pltpu.sync_copy(ctile_v, mat_hbm.at[wid + SPARE0])

    @pl.when((core == 1) & (wid == 0))
    def _():
        process(wid + HS, False)

    plsc.subcore_barrier()

    @pl.when((core == 0) & (wid == 0))
    def _():
        pltpu.sync_copy(bias_hbm, bias_v.at[pl.ds(0, 1)])
        pltpu.sync_copy(mat_hbm.at[pl.ds(SPARE0, NT)], sall_v)
        tot = jnp.zeros((16,), jnp.float32)
        for s in range(NT):
            tot = tot + sall_v[s, pl.ds(0, 16)]
        total = tot[0] + bias_v[pl.ds(0, 16)][0]
        res_v[pl.ds(0, 16)] = jnp.full((16,), total, jnp.float32)
        pltpu.sync_copy(res_v.at[pl.ds(0, 1)], res_hbm)


def kernel(items_emb, feature_emb, user_emb, Bias, ui_pair, feature_index,
           preference_index):
    del feature_index
    mesh = plsc.VectorSubcoreMesh(core_axis_name="c", subcore_axis_name="s",
                                  num_cores=2)
    fn = pl.kernel(
        _fm_body,
        mesh=mesh,
        out_type=(
            jax.ShapeDtypeStruct((SPARE0 + NT, MAT_W), jnp.float32),
            jax.ShapeDtypeStruct((1,), jnp.float32),
        ),
        scratch_types=[
            pltpu.VMEM((208,), jnp.int32),
            pltpu.VMEM((16,), jnp.int32),
            pltpu.VMEM((80,), jnp.float32),
            pltpu.VMEM((64,), jnp.float32),
            pltpu.VMEM((V_FEAT,), jnp.float32),
            pltpu.VMEM((MAT_W,), jnp.float32),
            pltpu.VMEM((MAT_W,), jnp.float32),
            pltpu.VMEM((NT, MAT_W), jnp.float32),
            pltpu.VMEM((16,), jnp.float32),
            pltpu.VMEM((16,), jnp.float32),
            pltpu.SemaphoreType.DMA,
        ],
        compiler_params=pltpu.CompilerParams(needs_layout_passes=False),
    )
    matT, res = fn(feature_emb.T, items_emb.T, user_emb, Bias, ui_pair,
                   preference_index)
    result = res.reshape(1, 1)
    mat = matT[:HS1, PREF0 - 2:PREF0 + L].T
    fb = mat[None, :, HS:]
    nz = mat[None, :, :HS]
    return (result, fb, nz)


# speedup vs baseline: 9.3589x; 1.0144x over previous
import jax
import jax.numpy as jnp
from jax import lax
from jax.experimental import pallas as pl
from jax.experimental.pallas import tpu as pltpu
from jax.experimental.pallas import tpu_sc as plsc

HS = 64
HS1 = HS + 1
L = 200
V_FEAT = 100000
V_ITEM = 1000000
NT = 16
MAT_W = 256
PREF0 = 16
SPARE0 = 72


def _fm_body(featT_hbm, itemsT_hbm, user_hbm, bias_hbm, ui_hbm, pref_hbm,
             mat_hbm, res_hbm,
             idx_v, uiv, ubuf, ibuf, rowstage, rowbuf, ctile_v, sall_v,
             bias_v, res_v, sem):
    core = lax.axis_index("c")
    wid = lax.axis_index("s")
    iota = lax.iota(jnp.int32, 16)

    pltpu.sync_copy(pref_hbm.at[0], idx_v.at[pl.ds(0, L)])
    c12 = idx_v[pl.ds(192, 16)]
    idx_v[pl.ds(192, 16)] = jnp.where(lax.iota(jnp.int32, 16) < 8, c12, 0)
    pltpu.sync_copy(ui_hbm.at[0], uiv.at[pl.ds(0, 2)])
    pltpu.sync_copy(user_hbm.at[0], ubuf.at[pl.ds(0, HS1)])
    i_item = uiv[pl.ds(0, 16)][1]
    woff = (i_item // 64) * 64
    lane_it = i_item - woff

    acc = jnp.zeros((16,), jnp.float32)
    res_v[pl.ds(0, 16)] = acc

    def process(d, with_contrib):
        h1 = pltpu.async_copy(featT_hbm.at[d], rowstage, sem)
        pltpu.sync_copy(itemsT_hbm.at[d, pl.ds(woff, 64)], ibuf)
        ue = jnp.float32(0.0)
        for k in range(5):
            uc = ubuf[pl.ds(16 * k, 16)]
            ue = ue + jnp.sum(jnp.where(iota + (16 * k) == d, uc, 0.0))
        ie = jnp.float32(0.0)
        for k in range(4):
            ic = ibuf[pl.ds(16 * k, 16)]
            ie = ie + jnp.sum(jnp.where(iota + (16 * k) == lane_it, ic, 0.0))
        hv = (jnp.where(iota == 14, ue, 0.0)
              + jnp.where(iota == 15, ie, 0.0))
        rowbuf[pl.ds(0, 16)] = hv
        h1.wait()
        psum = jnp.float32(0.0)
        for c in range(13):
            idxc = idx_v[pl.ds(16 * c, 16)]
            g = plsc.load_gather(rowstage, [idxc])
            valid = iota + (16 * c) < L
            psum = psum + jnp.sum(jnp.where(valid, g, 0.0))
            rowbuf[pl.ds(PREF0 + 16 * c, 16)] = g
        pltpu.sync_copy(rowbuf, mat_hbm.at[d])
        if with_contrib:
            contrib = ue * ie + (ue + ie) * psum
            cur = res_v[pl.ds(0, 16)]
            res_v[pl.ds(0, 16)] = cur + jnp.where(iota == 0, contrib, 0.0)

    @pl.when(core == 0)
    def _():
        for slot in range(4):
            process(wid + NT * slot, True)
        ctile_v[pl.ds(0, 16)] = res_v[pl.ds(0, 16)]
        zero16 = jnp.zeros((16,), jnp.float32)
        for k in range(1, MAT_W // 16):
            ctile_v[pl.ds(16 * k, 16)] = zero16
        pltpu.sync_copy(ctile_v, mat_hbm.at[wid + SPARE0])

    @pl.when((core == 1) & (wid == 0))
    def _():
        process(wid + HS, False)

    plsc.subcore_barrier()

    @pl.when((core == 0) & (wid == 0))
    def _():
        pltpu.sync_copy(bias_hbm, bias_v.at[pl.ds(0, 1)])
        pltpu.sync_copy(mat_hbm.at[pl.ds(SPARE0, NT)], sall_v)
        tot = jnp.zeros((16,), jnp.float32)
        for s in range(NT):
            tot = tot + sall_v[s, pl.ds(0, 16)]
        total = tot[0] + bias_v[pl.ds(0, 16)][0]
        res_v[pl.ds(0, 16)] = jnp.full((16,), total, jnp.float32)
        pltpu.sync_copy(res_v.at[pl.ds(0, 1)], res_hbm)


def kernel(items_emb, feature_emb, user_emb, Bias, ui_pair, feature_index,
           preference_index):
    del feature_index
    mesh = plsc.VectorSubcoreMesh(core_axis_name="c", subcore_axis_name="s",
                                  num_cores=2)
    fn = pl.kernel(
        _fm_body,
        mesh=mesh,
        out_type=(
            jax.ShapeDtypeStruct((SPARE0 + NT, MAT_W), jnp.float32),
            jax.ShapeDtypeStruct((1,), jnp.float32),
        ),
        scratch_types=[
            pltpu.VMEM((208,), jnp.int32),
            pltpu.VMEM((16,), jnp.int32),
            pltpu.VMEM((80,), jnp.float32),
            pltpu.VMEM((64,), jnp.float32),
            pltpu.VMEM((V_FEAT,), jnp.float32),
            pltpu.VMEM((MAT_W,), jnp.float32),
            pltpu.VMEM((MAT_W,), jnp.float32),
            pltpu.VMEM((NT, MAT_W), jnp.float32),
            pltpu.VMEM((16,), jnp.float32),
            pltpu.VMEM((16,), jnp.float32),
            pltpu.SemaphoreType.DMA,
        ],
        compiler_params=pltpu.CompilerParams(needs_layout_passes=False),
    )
    matT, res = fn(feature_emb.T, items_emb.T, user_emb, Bias, ui_pair,
                   preference_index)
    result = res.reshape(1, 1)
    mat = matT[:HS1, PREF0 - 2:PREF0 + L].T
    fb = mat[None, :, HS:]
    nz = mat[None, :, :HS]
    return (result, fb, nz)
